# Initial kernel scaffold; baseline (speedup 1.0000x reference)
import jax
import jax.numpy as jnp
from jax.experimental import pallas as pl


def kernel(x, edge_index, batch, Wd0, bd0, Wd1, bd1, Wd2, bd2, p0, p1, Wu0, bu0, Wu1, bu1, bn_g, bn_b, O0W, O0b, O1W, O1b, O2W, O2b):
    raise NotImplementedError("write your pallas kernel here")


# speedup vs baseline: 1.9324x; 1.9324x over previous
import functools
import math

import jax
import jax.numpy as jnp
from jax import lax
from jax.experimental import pallas as pl
from jax.experimental.pallas import tpu as pltpu

N = 10000
E = 160000
HID = 32
K1 = 2000
K2 = 400
F32 = jnp.float32


def _prep1_body(x_ref, w_ref, rs_ref, sc_ref, xw_ref, tab_ref, dis_ref, cor_ref):
    x = x_ref[...]
    xw = jnp.dot(x, w_ref[...].T, preferred_element_type=F32)
    rs = rs_ref[...]
    sc = sc_ref[...]
    d = jnp.where(sc > 0, sc, 2.0)
    deg = rs - sc + d
    dis = lax.rsqrt(deg)
    xw_ref[...] = xw
    tab_ref[...] = dis * xw
    dis_ref[...] = dis
    cor_ref[...] = (d - sc) * dis * dis


def tc_prep1(x, Wd0, rowsum, selfcnt):
    return pl.pallas_call(
        _prep1_body,
        out_shape=[
            jax.ShapeDtypeStruct((N, HID), F32),
            jax.ShapeDtypeStruct((N, HID), F32),
            jax.ShapeDtypeStruct((N, 1), F32),
            jax.ShapeDtypeStruct((N, 1), F32),
        ],
    )(x, Wd0, rowsum.reshape(N, 1), selfcnt.reshape(N, 1))


def _post1_body(agg_ref, dis_ref, cor_ref, xw_ref, b_ref, p_ref, h_ref, s_ref):
    agg = agg_ref[0] + agg_ref[1]
    y = dis_ref[...] * agg + cor_ref[...] * xw_ref[...] + b_ref[...]
    h = jnp.maximum(y, 0.0)
    h_ref[...] = h
    p = p_ref[...]
    pn = p * lax.rsqrt(jnp.sum(p * p))
    s_ref[...] = jnp.tanh(jnp.dot(h, pn.T, preferred_element_type=F32))


def tc_post1(aggp, dis, cor, xw, b, p):
    return pl.pallas_call(
        _post1_body,
        out_shape=[
            jax.ShapeDtypeStruct((N, HID), F32),
            jax.ShapeDtypeStruct((N, 1), F32),
        ],
    )(aggp, dis, cor, xw, b.reshape(1, HID), p.reshape(1, HID))


def _gather_body(h_ref, perm_ref, val_ref, out_ref, *, k, n, d, chunk, scale):
    acc = jnp.zeros((k, d), F32)
    permc = perm_ref[...].astype(jnp.int32)

    def body(t, acc):
        base = pl.multiple_of(t * chunk, 8)
        hc = h_ref[pl.ds(base, chunk), :]
        cols = lax.broadcasted_iota(jnp.int32, (k, chunk), 1) + t * chunk
        oh = jnp.where(cols == permc, 1.0, 0.0).astype(F32)
        return acc + jnp.dot(oh, hc, preferred_element_type=F32)

    acc = lax.fori_loop(0, n // chunk, body, acc)
    if scale:
        acc = acc * val_ref[...]
    out_ref[...] = acc


def tc_gather(h, perm, vals, k, scale=True):
    n, d = h.shape
    chunk = 400
    body = functools.partial(_gather_body, k=k, n=n, d=d, chunk=chunk, scale=scale)
    return pl.pallas_call(
        body,
        out_shape=jax.ShapeDtypeStruct((k, d), F32),
    )(h, perm.reshape(k, 1).astype(jnp.int32), vals.reshape(k, 1))


def _gathercols_body(m_ref, perm_ref, out_ref, *, k, n, d):
    permr = perm_ref[...].astype(jnp.int32)
    rows = lax.broadcasted_iota(jnp.int32, (n, k), 0)
    oh = jnp.where(rows == permr, 1.0, 0.0).astype(F32)
    out_ref[...] = jnp.dot(m_ref[...], oh, preferred_element_type=F32)


def tc_gather_cols(m, perm, k):
    d, n = m.shape
    body = functools.partial(_gathercols_body, k=k, n=n, d=d)
    return pl.pallas_call(
        body,
        out_shape=jax.ShapeDtypeStruct((d, k), F32),
    )(m, perm.reshape(1, k).astype(jnp.int32))


def _scatter_body(h_ref, perm_ref, out_ref, *, k, n, d, chunk):
    permr = perm_ref[...].astype(jnp.int32)

    def body(t, _):
        base = pl.multiple_of(t * chunk, 8)
        rows = lax.broadcasted_iota(jnp.int32, (chunk, k), 0) + t * chunk
        oh = jnp.where(rows == permr, 1.0, 0.0).astype(F32)
        out_ref[pl.ds(base, chunk), :] = jnp.dot(
            oh, h_ref[...], preferred_element_type=F32)
        return 0

    lax.fori_loop(0, n // chunk, body, 0)


def tc_scatter(h, perm, n):
    k, d = h.shape
    chunk = 400
    body = functools.partial(_scatter_body, k=k, n=n, d=d, chunk=chunk)
    return pl.pallas_call(
        body,
        out_shape=jax.ShapeDtypeStruct((n, d), F32),
    )(h, perm.reshape(1, k).astype(jnp.int32))


def _mm_body(a_ref, b_ref, s_ref, o_ref, *, bm, bk, nsteps, add_eye, s_scale,
             zero_diag, C):
    i = pl.program_id(0)
    t = pl.program_id(1)
    a = a_ref[...]
    if add_eye:
        rows = lax.broadcasted_iota(jnp.int32, (bm, bk), 0) + i * bm
        cols = lax.broadcasted_iota(jnp.int32, (bm, bk), 1) + t * bk
        a = a + jnp.where(rows == cols, 1.0, 0.0).astype(F32)
        b = b_ref[...]
        brows = lax.broadcasted_iota(jnp.int32, (bk, C), 0) + t * bk
        bcols = lax.broadcasted_iota(jnp.int32, (bk, C), 1)
        b = b + jnp.where(brows == bcols, 1.0, 0.0).astype(F32)
    else:
        b = b_ref[...]

    @pl.when(t == 0)
    def _init():
        o_ref[...] = jnp.zeros_like(o_ref)

    o_ref[...] += jnp.dot(a, b, preferred_element_type=F32)

    @pl.when(t == nsteps - 1)
    def _fin():
        o = o_ref[...]
        if s_scale != 0.0:
            o = o + s_scale * s_ref[...]
        if zero_diag:
            rows = lax.broadcasted_iota(jnp.int32, (bm, C), 0) + i * bm
            cols = lax.broadcasted_iota(jnp.int32, (bm, C), 1)
            o = jnp.where(rows == cols, 0.0, o)
        o_ref[...] = o


def tc_matmul(a, b, s=None, add_eye=False, s_scale=0.0, zero_diag=True,
              bm=256, bk=1024):
    M, K = a.shape
    K2_, C = b.shape
    nsteps = K // bk
    if s is None:
        s = jnp.zeros((1, 1), F32)
        s_spec = pl.BlockSpec((1, 1), lambda i, t: (0, 0))
    else:
        s_spec = pl.BlockSpec((bm, C), lambda i, t: (i, 0))
    body = functools.partial(_mm_body, bm=bm, bk=bk, nsteps=nsteps,
                             add_eye=add_eye, s_scale=s_scale,
                             zero_diag=zero_diag, C=C)
    return pl.pallas_call(
        body,
        grid=(M // bm, nsteps),
        in_specs=[
            pl.BlockSpec((bm, bk), lambda i, t: (i, t)),
            pl.BlockSpec((bk, C), lambda i, t: (t, 0)),
            s_spec,
        ],
        out_specs=pl.BlockSpec((bm, C), lambda i, t: (i, 0)),
        out_shape=jax.ShapeDtypeStruct((M, C), F32),
    )(a, b, s)


def _densegcn_body(a_ref, x1_ref, x2_ref, w_ref, b_ref, p_ref, h_ref, s_ref,
                   *, relu, use_x2, use_p):
    A = a_ref[...]
    xin = x1_ref[...]
    if use_x2:
        xin = xin + x2_ref[...]
    xw = jnp.dot(xin, w_ref[...].T, preferred_element_type=F32)
    deg = jnp.sum(A, axis=1, keepdims=True) + 2.0
    dis = lax.rsqrt(deg)
    y = dis * jnp.dot(A, dis * xw, preferred_element_type=F32)
    y = y + 2.0 * dis * dis * xw + b_ref[...]
    if relu:
        y = jnp.maximum(y, 0.0)
    h_ref[...] = y
    if use_p:
        p = p_ref[...]
        pn = p * lax.rsqrt(jnp.sum(p * p))
        s_ref[...] = jnp.tanh(jnp.dot(y, pn.T, preferred_element_type=F32))
    else:
        s_ref[...] = jnp.zeros_like(s_ref)


def tc_densegcn(A, x1, W, b, x2=None, p=None, relu=True):
    n = A.shape[0]
    use_x2 = x2 is not None
    use_p = p is not None
    if x2 is None:
        x2 = jnp.zeros((1, HID), F32)
    if p is None:
        p = jnp.zeros((1, HID), F32)
    body = functools.partial(_densegcn_body, relu=relu, use_x2=use_x2,
                             use_p=use_p)
    return pl.pallas_call(
        body,
        out_shape=[
            jax.ShapeDtypeStruct((n, HID), F32),
            jax.ShapeDtypeStruct((n, 1), F32),
        ],
    )(A, x1, x2, W.astype(F32), b.reshape(1, HID), p.reshape(1, HID))


def _prepup1_body(x0_ref, up_ref, w_ref, dis_ref, zw_ref, tab_ref):
    z = x0_ref[...] + up_ref[...]
    zw = jnp.dot(z, w_ref[...].T, preferred_element_type=F32)
    zw_ref[...] = zw
    tab_ref[...] = dis_ref[...] * zw


def tc_prepup1(x0, up, Wu1, dis):
    return pl.pallas_call(
        _prepup1_body,
        out_shape=[
            jax.ShapeDtypeStruct((N, HID), F32),
            jax.ShapeDtypeStruct((N, HID), F32),
        ],
    )(x0, up, Wu1, dis)


def _final_body(agg_ref, dis_ref, cor_ref, zw_ref, bu_ref, g_ref, bb_ref,
                w0_ref, b0_ref, w1_ref, b1_ref, w2_ref, b2_ref, o_ref):
    agg = agg_ref[0] + agg_ref[1]
    h = dis_ref[...] * agg + cor_ref[...] * zw_ref[...] + bu_ref[...]
    mu = jnp.mean(h, axis=0, keepdims=True)
    hc = h - mu
    var = jnp.mean(hc * hc, axis=0, keepdims=True)
    hb = hc * lax.rsqrt(var + 1e-5) * g_ref[...] + bb_ref[...]
    r = jnp.dot(hb, w0_ref[...].T, preferred_element_type=F32) + b0_ref[...]
    r = jnp.where(r > 0, r, 0.01 * r)
    r = jnp.dot(r, w1_ref[...].T, preferred_element_type=F32) + b1_ref[...]
    r = jnp.where(r > 0, r, 0.01 * r)
    r = jnp.dot(r, w2_ref[...].T, preferred_element_type=F32) + b2_ref[...]
    col = lax.broadcasted_iota(jnp.int32, r.shape, 1)
    o_ref[...] = jnp.where(col == 0, jax.nn.sigmoid(r), r)


def tc_final(aggp, dis, cor, zw, bu1, bn_g, bn_b, O0W, O0b, O1W, O1b, O2W, O2b):
    return pl.pallas_call(
        _final_body,
        out_shape=jax.ShapeDtypeStruct((N, 4), F32),
    )(aggp, dis, cor, zw, bu1.reshape(1, HID), bn_g.reshape(1, HID),
      bn_b.reshape(1, HID), O0W, O0b.reshape(1, HID), O1W,
      O1b.reshape(1, HID), O2W, O2b.reshape(1, 4))


def sc_degree(src, dst):
    ones = jnp.ones((E,), F32)
    rowsum = jnp.zeros((N,), F32).at[dst].add(ones)
    selfcnt = jnp.zeros((N,), F32).at[dst].add(
        jnp.where(src == dst, 1.0, 0.0))
    return rowsum, selfcnt


def sc_edge_agg(src, dst, table):
    agg = jnp.zeros((N, HID), F32).at[dst].add(table[src])
    return jnp.stack([agg, jnp.zeros((N, HID), F32)])


def sc_scat(src, dst, inv1, R, C, map_row, map_col):
    r = inv1[dst] if map_row else dst
    c = inv1[src] if map_col else src
    valid = (src != dst)
    if map_row:
        valid &= r >= 0
    if map_col:
        valid &= c >= 0
    return jnp.zeros((R, C), F32).at[r, c].add(jnp.where(valid, 1.0, 0.0))


def kernel(x, edge_index, batch, Wd0, bd0, Wd1, bd1, Wd2, bd2, p0, p1,
           Wu0, bu0, Wu1, bu1, bn_g, bn_b, O0W, O0b, O1W, O1b, O2W, O2b):
    src = edge_index[0].astype(jnp.int32)
    dst = edge_index[1].astype(jnp.int32)

    rowsum, selfcnt = sc_degree(src, dst)
    xw, table1, dis0, cor0 = tc_prep1(x, Wd0, rowsum, selfcnt)
    agg1 = sc_edge_agg(src, dst, table1)
    h, score1 = tc_post1(agg1, dis0, cor0, xw, bd0, p0)

    vals1, perm1 = lax.top_k(score1[:, 0], K1)
    hp = tc_gather(h, perm1, vals1, K1)
    inv1f = tc_scatter(jnp.arange(1, K1 + 1, dtype=F32).reshape(K1, 1),
                       perm1, N)
    inv1 = inv1f[:, 0].astype(jnp.int32) - 1

    K1P, NP = 2048, 10240
    B1 = sc_scat(src, dst, inv1, K1P, NP, True, False)
    C1 = sc_scat(src, dst, inv1, NP, K1P, False, True)
    S1 = sc_scat(src, dst, inv1, K1P, K1P, True, True)
    A1pp = tc_matmul(B1, C1, s=S1, s_scale=2.0, zero_diag=True,
                     bm=256, bk=1024)
    A1p = A1pp[:K1, :K1]

    h1, score2 = tc_densegcn(A1p, hp, Wd1, bd1, p=p1)
    vals2, perm2 = lax.top_k(score2[:, 0], K2)
    h1p = tc_gather(h1, perm2, vals2, K2)

    A2 = tc_matmul(A1pp, A1pp, add_eye=True, zero_diag=True,
                   bm=256, bk=512)[:K1, :K1]
    A2rows = tc_gather(A2, perm2, vals2, K2, scale=False)
    A2p = tc_gather_cols(A2rows, perm2, K2)

    h2, _ = tc_densegcn(A2p, h1p, Wd2, bd2)
    up2 = tc_scatter(h2, perm2, K1)
    hu0, _ = tc_densegcn(A1p, h1, Wu0, bu0, x2=up2)

    up1 = tc_scatter(hu0, perm1, N)
    zw, table2 = tc_prepup1(h, up1, Wu1, dis0)
    agg2 = sc_edge_agg(src, dst, table2)
    return tc_final(agg2, dis0, cor0, zw, bu1, bn_g, bn_b,
                    O0W, O0b, O1W, O1b, O2W, O2b)


# speedup vs baseline: 1.9349x; 1.0013x over previous
import functools
import math

import jax
import jax.numpy as jnp
from jax import lax
from jax.experimental import pallas as pl
from jax.experimental.pallas import tpu as pltpu

N = 10000
E = 160000
HID = 32
K1 = 2000
K2 = 400
F32 = jnp.float32


def _prep1_body(x_ref, w_ref, rs_ref, sc_ref, xw_ref, tab_ref, dis_ref, cor_ref):
    x = x_ref[...]
    xw = jnp.dot(x, w_ref[...].T, preferred_element_type=F32)
    rs = rs_ref[...]
    sc = sc_ref[...]
    d = jnp.where(sc > 0, sc, 2.0)
    deg = rs - sc + d
    dis = lax.rsqrt(deg)
    xw_ref[...] = xw
    tab_ref[:N, :] = dis * xw
    tab_ref[N:, :] = jnp.zeros((NPAD - N, HID), F32)
    dis_ref[...] = dis
    cor_ref[...] = (d - sc) * dis * dis


def tc_prep1(x, Wd0, rowsum, selfcnt):
    return pl.pallas_call(
        _prep1_body,
        out_shape=[
            jax.ShapeDtypeStruct((N, HID), F32),
            jax.ShapeDtypeStruct((NPAD, HID), F32),
            jax.ShapeDtypeStruct((N, 1), F32),
            jax.ShapeDtypeStruct((N, 1), F32),
        ],
    )(x, Wd0, rowsum.reshape(N, 1), selfcnt.reshape(N, 1))


def _post1_body(agg_ref, dis_ref, cor_ref, xw_ref, b_ref, p_ref, h_ref, s_ref):
    agg = agg_ref[...]
    y = dis_ref[...] * agg + cor_ref[...] * xw_ref[...] + b_ref[...]
    h = jnp.maximum(y, 0.0)
    h_ref[...] = h
    p = p_ref[...]
    pn = p * lax.rsqrt(jnp.sum(p * p))
    s_ref[...] = jnp.tanh(jnp.dot(h, pn.T, preferred_element_type=F32))


def tc_post1(aggp, dis, cor, xw, b, p):
    return pl.pallas_call(
        _post1_body,
        out_shape=[
            jax.ShapeDtypeStruct((N, HID), F32),
            jax.ShapeDtypeStruct((N, 1), F32),
        ],
    )(aggp, dis, cor, xw, b.reshape(1, HID), p.reshape(1, HID))


def _gather_body(h_ref, perm_ref, val_ref, out_ref, *, k, n, d, chunk, scale):
    acc = jnp.zeros((k, d), F32)
    permc = perm_ref[...].astype(jnp.int32)

    def body(t, acc):
        base = pl.multiple_of(t * chunk, 8)
        hc = h_ref[pl.ds(base, chunk), :]
        cols = lax.broadcasted_iota(jnp.int32, (k, chunk), 1) + t * chunk
        oh = jnp.where(cols == permc, 1.0, 0.0).astype(F32)
        return acc + jnp.dot(oh, hc, preferred_element_type=F32)

    acc = lax.fori_loop(0, n // chunk, body, acc)
    if scale:
        acc = acc * val_ref[...]
    out_ref[...] = acc


def tc_gather(h, perm, vals, k, scale=True):
    n, d = h.shape
    chunk = 400
    body = functools.partial(_gather_body, k=k, n=n, d=d, chunk=chunk, scale=scale)
    return pl.pallas_call(
        body,
        out_shape=jax.ShapeDtypeStruct((k, d), F32),
    )(h, perm.reshape(k, 1).astype(jnp.int32), vals.reshape(k, 1))


def _gathercols_body(m_ref, perm_ref, out_ref, *, k, n, d):
    permr = perm_ref[...].astype(jnp.int32)
    rows = lax.broadcasted_iota(jnp.int32, (n, k), 0)
    oh = jnp.where(rows == permr, 1.0, 0.0).astype(F32)
    out_ref[...] = jnp.dot(m_ref[...], oh, preferred_element_type=F32)


def tc_gather_cols(m, perm, k):
    d, n = m.shape
    body = functools.partial(_gathercols_body, k=k, n=n, d=d)
    return pl.pallas_call(
        body,
        out_shape=jax.ShapeDtypeStruct((d, k), F32),
    )(m, perm.reshape(1, k).astype(jnp.int32))


def _scatter_body(h_ref, perm_ref, out_ref, *, k, n, d, chunk):
    permr = perm_ref[...].astype(jnp.int32)

    def body(t, _):
        base = pl.multiple_of(t * chunk, 8)
        rows = lax.broadcasted_iota(jnp.int32, (chunk, k), 0) + t * chunk
        oh = jnp.where(rows == permr, 1.0, 0.0).astype(F32)
        out_ref[pl.ds(base, chunk), :] = jnp.dot(
            oh, h_ref[...], preferred_element_type=F32)
        return 0

    lax.fori_loop(0, n // chunk, body, 0)


def tc_scatter(h, perm, n):
    k, d = h.shape
    chunk = 400
    body = functools.partial(_scatter_body, k=k, n=n, d=d, chunk=chunk)
    return pl.pallas_call(
        body,
        out_shape=jax.ShapeDtypeStruct((n, d), F32),
    )(h, perm.reshape(1, k).astype(jnp.int32))


def _mm_body(a_ref, b_ref, s_ref, o_ref, *, bm, bk, nsteps, add_eye, s_scale,
             zero_diag, C):
    i = pl.program_id(0)
    t = pl.program_id(1)
    a = a_ref[...]
    if add_eye:
        rows = lax.broadcasted_iota(jnp.int32, (bm, bk), 0) + i * bm
        cols = lax.broadcasted_iota(jnp.int32, (bm, bk), 1) + t * bk
        a = a + jnp.where(rows == cols, 1.0, 0.0).astype(F32)
        b = b_ref[...]
        brows = lax.broadcasted_iota(jnp.int32, (bk, C), 0) + t * bk
        bcols = lax.broadcasted_iota(jnp.int32, (bk, C), 1)
        b = b + jnp.where(brows == bcols, 1.0, 0.0).astype(F32)
    else:
        b = b_ref[...]

    @pl.when(t == 0)
    def _init():
        o_ref[...] = jnp.zeros_like(o_ref)

    o_ref[...] += jnp.dot(a, b, preferred_element_type=F32)

    @pl.when(t == nsteps - 1)
    def _fin():
        o = o_ref[...]
        if s_scale != 0.0:
            o = o + s_scale * s_ref[...]
        if zero_diag:
            rows = lax.broadcasted_iota(jnp.int32, (bm, C), 0) + i * bm
            cols = lax.broadcasted_iota(jnp.int32, (bm, C), 1)
            o = jnp.where(rows == cols, 0.0, o)
        o_ref[...] = o


def tc_matmul(a, b, s=None, add_eye=False, s_scale=0.0, zero_diag=True,
              bm=256, bk=1024):
    M, K = a.shape
    K2_, C = b.shape
    nsteps = K // bk
    if s is None:
        s = jnp.zeros((1, 1), F32)
        s_spec = pl.BlockSpec((1, 1), lambda i, t: (0, 0))
    else:
        s_spec = pl.BlockSpec((bm, C), lambda i, t: (i, 0))
    body = functools.partial(_mm_body, bm=bm, bk=bk, nsteps=nsteps,
                             add_eye=add_eye, s_scale=s_scale,
                             zero_diag=zero_diag, C=C)
    return pl.pallas_call(
        body,
        grid=(M // bm, nsteps),
        in_specs=[
            pl.BlockSpec((bm, bk), lambda i, t: (i, t)),
            pl.BlockSpec((bk, C), lambda i, t: (t, 0)),
            s_spec,
        ],
        out_specs=pl.BlockSpec((bm, C), lambda i, t: (i, 0)),
        out_shape=jax.ShapeDtypeStruct((M, C), F32),
    )(a, b, s)


def _densegcn_body(a_ref, x1_ref, x2_ref, w_ref, b_ref, p_ref, h_ref, s_ref,
                   *, relu, use_x2, use_p):
    A = a_ref[...]
    xin = x1_ref[...]
    if use_x2:
        xin = xin + x2_ref[...]
    xw = jnp.dot(xin, w_ref[...].T, preferred_element_type=F32)
    deg = jnp.sum(A, axis=1, keepdims=True) + 2.0
    dis = lax.rsqrt(deg)
    y = dis * jnp.dot(A, dis * xw, preferred_element_type=F32)
    y = y + 2.0 * dis * dis * xw + b_ref[...]
    if relu:
        y = jnp.maximum(y, 0.0)
    h_ref[...] = y
    if use_p:
        p = p_ref[...]
        pn = p * lax.rsqrt(jnp.sum(p * p))
        s_ref[...] = jnp.tanh(jnp.dot(y, pn.T, preferred_element_type=F32))
    else:
        s_ref[...] = jnp.zeros_like(s_ref)


def tc_densegcn(A, x1, W, b, x2=None, p=None, relu=True):
    n = A.shape[0]
    use_x2 = x2 is not None
    use_p = p is not None
    if x2 is None:
        x2 = jnp.zeros((1, HID), F32)
    if p is None:
        p = jnp.zeros((1, HID), F32)
    body = functools.partial(_densegcn_body, relu=relu, use_x2=use_x2,
                             use_p=use_p)
    return pl.pallas_call(
        body,
        out_shape=[
            jax.ShapeDtypeStruct((n, HID), F32),
            jax.ShapeDtypeStruct((n, 1), F32),
        ],
    )(A, x1, x2, W.astype(F32), b.reshape(1, HID), p.reshape(1, HID))


def _prepup1_body(x0_ref, up_ref, w_ref, dis_ref, zw_ref, tab_ref):
    z = x0_ref[...] + up_ref[...]
    zw = jnp.dot(z, w_ref[...].T, preferred_element_type=F32)
    zw_ref[...] = zw
    tab_ref[:N, :] = dis_ref[...] * zw
    tab_ref[N:, :] = jnp.zeros((NPAD - N, HID), F32)


def tc_prepup1(x0, up, Wu1, dis):
    return pl.pallas_call(
        _prepup1_body,
        out_shape=[
            jax.ShapeDtypeStruct((N, HID), F32),
            jax.ShapeDtypeStruct((NPAD, HID), F32),
        ],
    )(x0, up, Wu1, dis)


def _final_body(agg_ref, dis_ref, cor_ref, zw_ref, bu_ref, g_ref, bb_ref,
                w0_ref, b0_ref, w1_ref, b1_ref, w2_ref, b2_ref, o_ref):
    agg = agg_ref[...]
    h = dis_ref[...] * agg + cor_ref[...] * zw_ref[...] + bu_ref[...]
    mu = jnp.mean(h, axis=0, keepdims=True)
    hc = h - mu
    var = jnp.mean(hc * hc, axis=0, keepdims=True)
    hb = hc * lax.rsqrt(var + 1e-5) * g_ref[...] + bb_ref[...]
    r = jnp.dot(hb, w0_ref[...].T, preferred_element_type=F32) + b0_ref[...]
    r = jnp.where(r > 0, r, 0.01 * r)
    r = jnp.dot(r, w1_ref[...].T, preferred_element_type=F32) + b1_ref[...]
    r = jnp.where(r > 0, r, 0.01 * r)
    r = jnp.dot(r, w2_ref[...].T, preferred_element_type=F32) + b2_ref[...]
    col = lax.broadcasted_iota(jnp.int32, r.shape, 1)
    o_ref[...] = jnp.where(col == 0, jax.nn.sigmoid(r), r)


def tc_final(aggp, dis, cor, zw, bu1, bn_g, bn_b, O0W, O0b, O1W, O1b, O2W, O2b):
    return pl.pallas_call(
        _final_body,
        out_shape=jax.ShapeDtypeStruct((N, 4), F32),
    )(aggp, dis, cor, zw, bu1.reshape(1, HID), bn_g.reshape(1, HID),
      bn_b.reshape(1, HID), O0W, O0b.reshape(1, HID), O1W,
      O1b.reshape(1, HID), O2W, O2b.reshape(1, 4))


NPAD = 10240


def sc_scat(src, dst, inv1, R, C, map_row, map_col):
    r = inv1[dst] if map_row else dst
    c = inv1[src] if map_col else src
    valid = (src != dst)
    if map_row:
        valid &= r >= 0
    if map_col:
        valid &= c >= 0
    return jnp.zeros((R, C), F32).at[r, c].add(jnp.where(valid, 1.0, 0.0))


def kernel(x, edge_index, batch, Wd0, bd0, Wd1, bd1, Wd2, bd2, p0, p1,
           Wu0, bu0, Wu1, bu1, bn_g, bn_b, O0W, O0b, O1W, O1b, O2W, O2b):
    src = edge_index[0].astype(jnp.int32)
    dst = edge_index[1].astype(jnp.int32)

    rowsum = jnp.zeros((N,), F32).at[dst].add(1.0)
    selfcnt = jnp.zeros((N,), F32).at[dst].add(
        jnp.where(src == dst, 1.0, 0.0))
    xw, table1, dis0, cor0 = tc_prep1(x, Wd0, rowsum, selfcnt)
    agg1 = jnp.zeros((N, HID), F32).at[dst].add(table1[src])
    h, score1 = tc_post1(agg1, dis0, cor0, xw, bd0, p0)

    vals1, perm1 = lax.top_k(score1[:, 0], K1)
    hp = tc_gather(h, perm1, vals1, K1)
    inv1f = tc_scatter(jnp.arange(1, K1 + 1, dtype=F32).reshape(K1, 1),
                       perm1, N)
    inv1 = inv1f[:, 0].astype(jnp.int32) - 1

    K1P, NP = 2048, 10240
    B1 = sc_scat(src, dst, inv1, K1P, NP, True, False)
    C1 = sc_scat(src, dst, inv1, NP, K1P, False, True)
    S1 = sc_scat(src, dst, inv1, K1P, K1P, True, True)
    A1pp = tc_matmul(B1, C1, s=S1, s_scale=2.0, zero_diag=True,
                     bm=256, bk=1024)
    A1p = A1pp[:K1, :K1]

    h1, score2 = tc_densegcn(A1p, hp, Wd1, bd1, p=p1)
    vals2, perm2 = lax.top_k(score2[:, 0], K2)
    h1p = tc_gather(h1, perm2, vals2, K2)

    A2 = tc_matmul(A1pp, A1pp, add_eye=True, zero_diag=True,
                   bm=256, bk=512)[:K1, :K1]
    A2rows = tc_gather(A2, perm2, vals2, K2, scale=False)
    A2p = tc_gather_cols(A2rows, perm2, K2)

    h2, _ = tc_densegcn(A2p, h1p, Wd2, bd2)
    up2 = tc_scatter(h2, perm2, K1)
    hu0, _ = tc_densegcn(A1p, h1, Wu0, bu0, x2=up2)

    up1 = tc_scatter(hu0, perm1, N)
    zw, table2 = tc_prepup1(h, up1, Wu1, dis0)
    agg2 = jnp.zeros((N, HID), F32).at[dst].add(table2[src])
    return tc_final(agg2, dis0, cor0, zw, bu1, bn_g, bn_b,
                    O0W, O0b, O1W, O1b, O2W, O2b)
